# baseline (speedup 1.0000x reference)
import functools

import jax
import jax.numpy as jnp
from jax import lax
from jax.experimental import pallas as pl
from jax.experimental.pallas import tpu as pltpu
from jax.experimental.pallas import tpu_sc as plsc

N = 10000
E = 320000
F = 128
B = 1024
K = 128
NCHUNK = E // K
NP = 10112
NC = 2
NS = 16
NW = NC * NS
T = 80
EPAD = T * NW * K
ROWS_PER_SUB = NP // NS

_mesh = plsc.VectorSubcoreMesh(core_axis_name="c", subcore_axis_name="s")


def _wid():
    return lax.axis_index("s") * NC + lax.axis_index("c")


@functools.partial(
    pl.kernel,
    out_type=jax.ShapeDtypeStruct((NC, NP, F), jnp.float32),
    mesh=_mesh,
    scratch_types=[
        pltpu.VMEM_SHARED((NP, F), jnp.float32),
        pltpu.VMEM((K, F), jnp.float32),
        pltpu.VMEM((K,), jnp.int32),
        pltpu.VMEM((K,), jnp.int32),
        pltpu.SemaphoreType.DMA,
        pltpu.SemaphoreType.DMA,
    ],
)
def _deg_kernel(dsta, dstb, dstc, zeros_hbm, m0, m1, m2, out,
                shd, mv, dv0, dv1, s0, s1):
    c = lax.axis_index("c")
    s = lax.axis_index("s")
    wid = _wid()
    dsts = [dsta, dstb, dstc]
    ms = [m0, m1, m2]
    dvs = [dv0, dv1]
    sems = [s0, s1]
    for j in range(5):
        cz = s + NS * j

        @pl.when(cz < NP // K)
        def _():
            pltpu.sync_copy(zeros_hbm, shd.at[pl.ds(cz * K, K)])

    plsc.subcore_barrier()
    base = wid * T
    for r in range(3):
        pltpu.sync_copy(ms[r], mv)
        pltpu.sync_copy(dsts[r].at[pl.ds(base * K, K)], dv0)

        def body(j0, carry):
            for b in range(2):
                j = 2 * j0 + b

                @pl.when(j >= 1)
                def _():
                    pltpu.make_async_copy(mv, shd.at[dvs[1 - b]],
                                          sems[1 - b]).wait()

                pltpu.async_copy(mv, shd.at[dvs[b]], sems[b], add=True)

                @pl.when(j + 1 < T)
                def _():
                    pltpu.sync_copy(
                        dsts[r].at[pl.ds((base + j + 1) * K, K)], dvs[1 - b])

            return carry

        lax.fori_loop(0, T // 2, body, 0)
        pltpu.make_async_copy(mv, shd.at[dvs[(T - 1) % 2]],
                              sems[(T - 1) % 2]).wait()
    plsc.subcore_barrier()
    pltpu.sync_copy(shd.at[pl.ds(s * ROWS_PER_SUB, ROWS_PER_SUB)],
                    out.at[c, pl.ds(s * ROWS_PER_SUB, ROWS_PER_SUB)])


@functools.partial(
    pl.kernel,
    out_type=jax.ShapeDtypeStruct((NC, 3, NP, F), jnp.float32),
    mesh=_mesh,
    scratch_types=[
        pltpu.VMEM_SHARED((NP, F), jnp.float32),
        pltpu.VMEM((K, F), jnp.float32),
        pltpu.VMEM((K, F), jnp.float32),
        pltpu.VMEM((K,), jnp.int32),
        pltpu.VMEM((K,), jnp.int32),
        pltpu.VMEM((K,), jnp.int32),
        pltpu.VMEM((K,), jnp.int32),
        pltpu.SemaphoreType.DMA,
        pltpu.SemaphoreType.DMA,
        pltpu.SemaphoreType.DMA,
        pltpu.SemaphoreType.DMA,
    ],
)
def _scatter_kernel(hp0, hp1, hp2, srca, srcb, srcc, dsta, dstb, dstc,
                    zeros_hbm, out, acc, rows0, rows1, sv0, sv1, dv0, dv1,
                    sg0, sg1, ss0, ss1):
    c = lax.axis_index("c")
    s = lax.axis_index("s")
    wid = _wid()
    hps = [hp0, hp1, hp2]
    srcs = [srca, srcb, srcc]
    dsts = [dsta, dstb, dstc]
    rows = [rows0, rows1]
    svs = [sv0, sv1]
    dvs = [dv0, dv1]
    sgs = [sg0, sg1]
    sss = [ss0, ss1]
    base = wid * T
    for r in range(3):
        for j in range(5):
            cz = s + NS * j

            @pl.when(cz < NP // K)
            def _():
                pltpu.sync_copy(zeros_hbm, acc.at[pl.ds(cz * K, K)])

        plsc.subcore_barrier()

        def body(j, carry):
            pltpu.sync_copy(srcs[r].at[pl.ds((base + j) * K, K)], sv0)
            pltpu.sync_copy(dsts[r].at[pl.ds((base + j) * K, K)], dv0)
            pltpu.async_copy(hps[r].at[sv0], rows0, sg0).wait()
            pltpu.sync_copy(rows0, acc.at[dv0], add=True)
            return carry

        lax.fori_loop(0, T, body, 0)
        plsc.subcore_barrier()
        pltpu.sync_copy(acc.at[pl.ds(s * ROWS_PER_SUB, ROWS_PER_SUB)],
                        out.at[c, r, pl.ds(s * ROWS_PER_SUB, ROWS_PER_SUB)])
        if r < 2:
            plsc.subcore_barrier()


@functools.partial(
    pl.kernel,
    out_type=jax.ShapeDtypeStruct((B, F), jnp.float32),
    mesh=_mesh,
    scratch_types=[
        pltpu.VMEM((B // NW,), jnp.int32),
        pltpu.VMEM((B // NW, F), jnp.float32),
        pltpu.SemaphoreType.DMA,
    ],
)
def _gather_kernel(g_hbm, idx_hbm, out, idx_v, rows_v, sem):
    wid = _wid()
    base = wid * (B // NW)
    pltpu.sync_copy(idx_hbm.at[pl.ds(base, B // NW)], idx_v)
    pltpu.async_copy(g_hbm.at[idx_v], rows_v, sem).wait()
    pltpu.sync_copy(rows_v, out.at[pl.ds(base, B // NW)])


_BR = 1000


def _c1_body(x, dga, dgb, W0, W1, W2,
             hp0, hp1, hp2, dv0, dv1, dv2):
    Ws = [W0, W1, W2]
    hps = [hp0, hp1, hp2]
    dvs = [dv0, dv1, dv2]
    xb = x[...]
    for r in range(3):
        deg = dga[0, :, r:r + 1] + dgb[0, :, r:r + 1] + 1.0
        dinv = lax.rsqrt(deg)
        h = jnp.dot(xb, Ws[r][...], preferred_element_type=jnp.float32)
        hps[r][...] = h * dinv
        dvs[r][...] = dinv


def _c1(x, degp, W1s):
    spec_x = pl.BlockSpec((_BR, F), lambda i: (i, 0))
    spec_deg = [pl.BlockSpec((1, _BR, F), (lambda i, c=c: (c, i, 0)))
                for c in range(2)]
    spec_w = pl.BlockSpec((F, F), lambda i: (0, 0))
    spec_o = pl.BlockSpec((_BR, F), lambda i: (i, 0))
    spec_dv = pl.BlockSpec((_BR, 1), lambda i: (i, 0))
    return pl.pallas_call(
        _c1_body,
        grid=(N // _BR,),
        in_specs=[spec_x] + spec_deg + [spec_w] * 3,
        out_specs=[spec_o] * 3 + [spec_dv] * 3,
        out_shape=[jax.ShapeDtypeStruct((N, F), jnp.float32)] * 3
        + [jax.ShapeDtypeStruct((N, 1), jnp.float32)] * 3,
    )(x, degp, degp, *W1s)


def _ec_body(a00, a01, a10, a11, a20, a21, hp0, hp1, hp2,
             dv0, dv1, dv2, b0, b1, b2, W0, W1, W2, o0, o1, o2):
    accs = [(a00, a01), (a10, a11), (a20, a21)]
    hps = [hp0, hp1, hp2]
    dvs = [dv0, dv1, dv2]
    bs = [b0, b1, b2]
    Ws = [W0, W1, W2]
    outs = [o0, o1, o2]
    h = None
    for r in range(3):
        acc = accs[r][0][0, 0] + accs[r][1][0, 0]
        full = (acc + hps[r][...]) * dvs[r][...] + bs[r][...]
        v = jnp.maximum(full, 0.0)
        h = v if h is None else jnp.maximum(h, v)
    for r in range(3):
        o = jnp.dot(h, Ws[r][...], preferred_element_type=jnp.float32)
        outs[r][...] = o * dvs[r][...]


def _acc_specs():
    return [pl.BlockSpec((1, 1, _BR, F),
                         (lambda i, c=c, r=r: (c, r, i, 0)))
            for r in range(3) for c in range(2)]


def _ec(accp, hps, dvs, bs, W2s):
    spec_h = pl.BlockSpec((_BR, F), lambda i: (i, 0))
    spec_dv = pl.BlockSpec((_BR, 1), lambda i: (i, 0))
    spec_b = pl.BlockSpec((1, F), lambda i: (0, 0))
    spec_w = pl.BlockSpec((F, F), lambda i: (0, 0))
    return pl.pallas_call(
        _ec_body,
        grid=(N // _BR,),
        in_specs=_acc_specs() + [spec_h] * 3 + [spec_dv] * 3
        + [spec_b] * 3 + [spec_w] * 3,
        out_specs=[spec_h] * 3,
        out_shape=[jax.ShapeDtypeStruct((N, F), jnp.float32)] * 3,
    )(accp, accp, accp, accp, accp, accp, *hps, *dvs, *bs, *W2s)


def _e2_body(a00, a01, a10, a11, a20, a21, hp0, hp1, hp2,
             dv0, dv1, dv2, b0, b1, b2, g):
    accs = [(a00, a01), (a10, a11), (a20, a21)]
    hps = [hp0, hp1, hp2]
    dvs = [dv0, dv1, dv2]
    bs = [b0, b1, b2]
    h = None
    for r in range(3):
        acc = accs[r][0][0, 0] + accs[r][1][0, 0]
        full = (acc + hps[r][...]) * dvs[r][...] + bs[r][...]
        v = jnp.maximum(full, 0.0)
        h = v if h is None else jnp.maximum(h, v)
    g[...] = h


def _e2(accp, hps, dvs, bs):
    spec_h = pl.BlockSpec((_BR, F), lambda i: (i, 0))
    spec_dv = pl.BlockSpec((_BR, 1), lambda i: (i, 0))
    spec_b = pl.BlockSpec((1, F), lambda i: (0, 0))
    return pl.pallas_call(
        _e2_body,
        grid=(N // _BR,),
        in_specs=_acc_specs() + [spec_h] * 3 + [spec_dv] * 3 + [spec_b] * 3,
        out_specs=spec_h,
        out_shape=jax.ShapeDtypeStruct((N, F), jnp.float32),
    )(accp, accp, accp, accp, accp, accp, *hps, *dvs, *bs)


def kernel(x, syntactic_edge_index, sequential_edge_index, semantic_edge_index,
           index, W1_syn, b1_syn, W1_seq, b1_seq, W1_sem, b1_sem,
           W2_syn, b2_syn, W2_seq, b2_seq, W2_sem, b2_sem):
    es = [syntactic_edge_index, sequential_edge_index, semantic_edge_index]
    pad_src = jnp.zeros((EPAD - E,), jnp.int32)
    pad_dst = N + jnp.arange(EPAD - E, dtype=jnp.int32) % (NP - N)
    srcs = [jnp.concatenate([e[0], pad_src]) for e in es]
    dsts = [jnp.concatenate([e[1], pad_dst]) for e in es]
    zeros128 = jnp.zeros((K, F), jnp.float32)
    lane = lax.broadcasted_iota(jnp.int32, (K, F), 1)
    masks = [(lane == r).astype(jnp.float32) for r in range(3)]

    degp = _deg_kernel(dsts[0], dsts[1], dsts[2], zeros128, *masks)

    W1s = [W1_syn, W1_seq, W1_sem]
    b1s = [b1_syn.reshape(1, F), b1_seq.reshape(1, F), b1_sem.reshape(1, F)]
    W2s = [W2_syn, W2_seq, W2_sem]
    b2s = [b2_syn.reshape(1, F), b2_seq.reshape(1, F), b2_sem.reshape(1, F)]

    hp0, hp1, hp2, dv0, dv1, dv2 = _c1(x, degp, W1s)

    acc1 = _scatter_kernel(hp0, hp1, hp2, *srcs, *dsts, zeros128)

    h2p = _ec(acc1, [hp0, hp1, hp2], [dv0, dv1, dv2], b1s, W2s)

    acc2 = _scatter_kernel(h2p[0], h2p[1], h2p[2], *srcs, *dsts, zeros128)

    g = _e2(acc2, h2p, [dv0, dv1, dv2], b2s)
    return _gather_kernel(g, index)


# speedup vs baseline: 2.1056x; 2.1056x over previous
import functools

import jax
import jax.numpy as jnp
from jax import lax
from jax.experimental import pallas as pl
from jax.experimental.pallas import tpu as pltpu
from jax.experimental.pallas import tpu_sc as plsc

N = 10000
E = 320000
F = 128
B = 1024
K = 128
NCHUNK = E // K
NP = 10112
NC = 2
NS = 16
NW = NC * NS
T = 80
EPAD = T * NW * K
ROWS_PER_SUB = NP // NS

_mesh = plsc.VectorSubcoreMesh(core_axis_name="c", subcore_axis_name="s")


def _wid():
    return lax.axis_index("s") * NC + lax.axis_index("c")


@functools.partial(
    pl.kernel,
    out_type=jax.ShapeDtypeStruct((NC, NP, F), jnp.float32),
    mesh=_mesh,
    scratch_types=[
        pltpu.VMEM_SHARED((NP, F), jnp.float32),
        pltpu.VMEM((K, F), jnp.float32),
        pltpu.VMEM((K,), jnp.int32),
        pltpu.VMEM((K,), jnp.int32),
        pltpu.SemaphoreType.DMA,
        pltpu.SemaphoreType.DMA,
    ],
)
def _deg_kernel(dsta, dstb, dstc, zeros_hbm, m0, m1, m2, out,
                shd, mv, dv0, dv1, s0, s1):
    c = lax.axis_index("c")
    s = lax.axis_index("s")
    wid = _wid()
    dsts = [dsta, dstb, dstc]
    ms = [m0, m1, m2]
    dvs = [dv0, dv1]
    sems = [s0, s1]
    for j in range(5):
        cz = s + NS * j

        @pl.when(cz < NP // K)
        def _():
            pltpu.sync_copy(zeros_hbm, shd.at[pl.ds(cz * K, K)])

    plsc.subcore_barrier()
    base = wid * T
    for r in range(3):
        pltpu.sync_copy(ms[r], mv)
        pltpu.sync_copy(dsts[r].at[pl.ds(base * K, K)], dv0)

        def body(j0, carry):
            for b in range(2):
                j = 2 * j0 + b

                @pl.when(j >= 1)
                def _():
                    pltpu.make_async_copy(mv, shd.at[dvs[1 - b]],
                                          sems[1 - b]).wait()

                pltpu.async_copy(mv, shd.at[dvs[b]], sems[b], add=True)

                @pl.when(j + 1 < T)
                def _():
                    pltpu.sync_copy(
                        dsts[r].at[pl.ds((base + j + 1) * K, K)], dvs[1 - b])

            return carry

        lax.fori_loop(0, T // 2, body, 0)
        pltpu.make_async_copy(mv, shd.at[dvs[(T - 1) % 2]],
                              sems[(T - 1) % 2]).wait()
    plsc.subcore_barrier()
    pltpu.sync_copy(shd.at[pl.ds(s * ROWS_PER_SUB, ROWS_PER_SUB)],
                    out.at[c, pl.ds(s * ROWS_PER_SUB, ROWS_PER_SUB)])


@functools.partial(
    pl.kernel,
    out_type=jax.ShapeDtypeStruct((NC, 3, NP, F), jnp.float32),
    mesh=_mesh,
    scratch_types=[
        pltpu.VMEM_SHARED((NP, F), jnp.float32),
        pltpu.VMEM((K, F), jnp.float32),
        pltpu.VMEM((K, F), jnp.float32),
        pltpu.VMEM((K,), jnp.int32),
        pltpu.VMEM((K,), jnp.int32),
        pltpu.VMEM((K,), jnp.int32),
        pltpu.VMEM((K,), jnp.int32),
        pltpu.SemaphoreType.DMA,
        pltpu.SemaphoreType.DMA,
        pltpu.SemaphoreType.DMA,
        pltpu.SemaphoreType.DMA,
    ],
)
def _scatter_kernel(hp0, hp1, hp2, srca, srcb, srcc, dsta, dstb, dstc,
                    zeros_hbm, out, acc, rows0, rows1, sv0, sv1, dv0, dv1,
                    sg0, sg1, ss0, ss1):
    c = lax.axis_index("c")
    s = lax.axis_index("s")
    wid = _wid()
    hps = [hp0, hp1, hp2]
    srcs = [srca, srcb, srcc]
    dsts = [dsta, dstb, dstc]
    rows = [rows0, rows1]
    svs = [sv0, sv1]
    dvs = [dv0, dv1]
    sgs = [sg0, sg1]
    sss = [ss0, ss1]
    base = wid * T
    for r in range(3):
        for j in range(5):
            cz = s + NS * j

            @pl.when(cz < NP // K)
            def _():
                pltpu.sync_copy(zeros_hbm, acc.at[pl.ds(cz * K, K)])

        plsc.subcore_barrier()

        def body(j, carry):
            pltpu.sync_copy(srcs[r].at[pl.ds((base + j) * K, K)], sv0)
            pltpu.sync_copy(dsts[r].at[pl.ds((base + j) * K, K)], dv0)
            pltpu.async_copy(hps[r].at[sv0], rows0, sg0).wait()
            pltpu.sync_copy(rows0, acc.at[dv0], add=True)
            return carry

        lax.fori_loop(0, T, body, 0)
        plsc.subcore_barrier()
        pltpu.sync_copy(acc.at[pl.ds(s * ROWS_PER_SUB, ROWS_PER_SUB)],
                        out.at[c, r, pl.ds(s * ROWS_PER_SUB, ROWS_PER_SUB)])
        if r < 2:
            plsc.subcore_barrier()


@functools.partial(
    pl.kernel,
    out_type=jax.ShapeDtypeStruct((B, F), jnp.float32),
    mesh=_mesh,
    scratch_types=[
        pltpu.VMEM((B // NW,), jnp.int32),
        pltpu.VMEM((B // NW, F), jnp.float32),
        pltpu.SemaphoreType.DMA,
    ],
)
def _gather_kernel(g_hbm, idx_hbm, out, idx_v, rows_v, sem):
    wid = _wid()
    base = wid * (B // NW)
    pltpu.sync_copy(idx_hbm.at[pl.ds(base, B // NW)], idx_v)
    pltpu.async_copy(g_hbm.at[idx_v], rows_v, sem).wait()
    pltpu.sync_copy(rows_v, out.at[pl.ds(base, B // NW)])


_BR = 1000


def _c1_body(x, dga, dgb, W0, W1, W2,
             hp0, hp1, hp2, dv0, dv1, dv2):
    Ws = [W0, W1, W2]
    hps = [hp0, hp1, hp2]
    dvs = [dv0, dv1, dv2]
    xb = x[...]
    for r in range(3):
        deg = dga[0, :, r:r + 1] + dgb[0, :, r:r + 1] + 1.0
        dinv = lax.rsqrt(deg)
        h = jnp.dot(xb, Ws[r][...], preferred_element_type=jnp.float32)
        hps[r][...] = h * dinv
        dvs[r][...] = dinv


def _c1(x, degp, W1s):
    spec_x = pl.BlockSpec((_BR, F), lambda i: (i, 0))
    spec_deg = [pl.BlockSpec((1, _BR, F), (lambda i, c=c: (c, i, 0)))
                for c in range(2)]
    spec_w = pl.BlockSpec((F, F), lambda i: (0, 0))
    spec_o = pl.BlockSpec((_BR, F), lambda i: (i, 0))
    spec_dv = pl.BlockSpec((_BR, 1), lambda i: (i, 0))
    return pl.pallas_call(
        _c1_body,
        grid=(N // _BR,),
        in_specs=[spec_x] + spec_deg + [spec_w] * 3,
        out_specs=[spec_o] * 3 + [spec_dv] * 3,
        out_shape=[jax.ShapeDtypeStruct((N, F), jnp.float32)] * 3
        + [jax.ShapeDtypeStruct((N, 1), jnp.float32)] * 3,
    )(x, degp, degp, *W1s)


def _ec_body(a00, a01, a10, a11, a20, a21, hp0, hp1, hp2,
             dv0, dv1, dv2, b0, b1, b2, W0, W1, W2, o0, o1, o2):
    accs = [(a00, a01), (a10, a11), (a20, a21)]
    hps = [hp0, hp1, hp2]
    dvs = [dv0, dv1, dv2]
    bs = [b0, b1, b2]
    Ws = [W0, W1, W2]
    outs = [o0, o1, o2]
    h = None
    for r in range(3):
        acc = accs[r][0][0, 0] + accs[r][1][0, 0]
        full = (acc + hps[r][...]) * dvs[r][...] + bs[r][...]
        v = jnp.maximum(full, 0.0)
        h = v if h is None else jnp.maximum(h, v)
    for r in range(3):
        o = jnp.dot(h, Ws[r][...], preferred_element_type=jnp.float32)
        outs[r][...] = o * dvs[r][...]


def _acc_specs():
    return [pl.BlockSpec((1, 1, _BR, F),
                         (lambda i, c=c, r=r: (c, r, i, 0)))
            for r in range(3) for c in range(2)]


def _ec(accp, hps, dvs, bs, W2s):
    spec_h = pl.BlockSpec((_BR, F), lambda i: (i, 0))
    spec_dv = pl.BlockSpec((_BR, 1), lambda i: (i, 0))
    spec_b = pl.BlockSpec((1, F), lambda i: (0, 0))
    spec_w = pl.BlockSpec((F, F), lambda i: (0, 0))
    return pl.pallas_call(
        _ec_body,
        grid=(N // _BR,),
        in_specs=_acc_specs() + [spec_h] * 3 + [spec_dv] * 3
        + [spec_b] * 3 + [spec_w] * 3,
        out_specs=[spec_h] * 3,
        out_shape=[jax.ShapeDtypeStruct((N, F), jnp.float32)] * 3,
    )(accp, accp, accp, accp, accp, accp, *hps, *dvs, *bs, *W2s)


def _e2_body(a00, a01, a10, a11, a20, a21, hp0, hp1, hp2,
             dv0, dv1, dv2, b0, b1, b2, g):
    accs = [(a00, a01), (a10, a11), (a20, a21)]
    hps = [hp0, hp1, hp2]
    dvs = [dv0, dv1, dv2]
    bs = [b0, b1, b2]
    h = None
    for r in range(3):
        acc = accs[r][0][0, 0] + accs[r][1][0, 0]
        full = (acc + hps[r][...]) * dvs[r][...] + bs[r][...]
        v = jnp.maximum(full, 0.0)
        h = v if h is None else jnp.maximum(h, v)
    g[...] = h


def _e2(accp, hps, dvs, bs):
    spec_h = pl.BlockSpec((_BR, F), lambda i: (i, 0))
    spec_dv = pl.BlockSpec((_BR, 1), lambda i: (i, 0))
    spec_b = pl.BlockSpec((1, F), lambda i: (0, 0))
    return pl.pallas_call(
        _e2_body,
        grid=(N // _BR,),
        in_specs=_acc_specs() + [spec_h] * 3 + [spec_dv] * 3 + [spec_b] * 3,
        out_specs=spec_h,
        out_shape=jax.ShapeDtypeStruct((N, F), jnp.float32),
    )(accp, accp, accp, accp, accp, accp, *hps, *dvs, *bs)


def kernel(x, syntactic_edge_index, sequential_edge_index, semantic_edge_index,
           index, W1_syn, b1_syn, W1_seq, b1_seq, W1_sem, b1_sem,
           W2_syn, b2_syn, W2_seq, b2_seq, W2_sem, b2_sem):
    es = [syntactic_edge_index, sequential_edge_index, semantic_edge_index]
    pad_src = jnp.arange(EPAD - E, dtype=jnp.int32) % N
    pad_dst = N + jnp.arange(EPAD - E, dtype=jnp.int32) % (NP - N)
    srcs = [jnp.concatenate([e[0], pad_src]) for e in es]
    dsts = [jnp.concatenate([e[1], pad_dst]) for e in es]
    zeros128 = jnp.zeros((K, F), jnp.float32)
    lane = lax.broadcasted_iota(jnp.int32, (K, F), 1)
    masks = [(lane == r).astype(jnp.float32) for r in range(3)]

    degp = _deg_kernel(dsts[0], dsts[1], dsts[2], zeros128, *masks)

    W1s = [W1_syn, W1_seq, W1_sem]
    b1s = [b1_syn.reshape(1, F), b1_seq.reshape(1, F), b1_sem.reshape(1, F)]
    W2s = [W2_syn, W2_seq, W2_sem]
    b2s = [b2_syn.reshape(1, F), b2_seq.reshape(1, F), b2_sem.reshape(1, F)]

    hp0, hp1, hp2, dv0, dv1, dv2 = _c1(x, degp, W1s)

    acc1 = _scatter_kernel(hp0, hp1, hp2, *srcs, *dsts, zeros128)

    h2p = _ec(acc1, [hp0, hp1, hp2], [dv0, dv1, dv2], b1s, W2s)

    acc2 = _scatter_kernel(h2p[0], h2p[1], h2p[2], *srcs, *dsts, zeros128)

    g = _e2(acc2, h2p, [dv0, dv1, dv2], b2s)
    return _gather_kernel(g, index)


# speedup vs baseline: 2.4822x; 1.1788x over previous
import functools

import jax
import jax.numpy as jnp
from jax import lax
from jax.experimental import pallas as pl
from jax.experimental.pallas import tpu as pltpu
from jax.experimental.pallas import tpu_sc as plsc

N = 10000
E = 320000
F = 128
B = 1024
K = 128
NCHUNK = E // K
NP = 10112
NC = 2
NS = 16
NW = NC * NS
T = 80
EPAD = T * NW * K
ROWS_PER_SUB = NP // NS

_mesh = plsc.VectorSubcoreMesh(core_axis_name="c", subcore_axis_name="s")


def _wid():
    return lax.axis_index("s") * NC + lax.axis_index("c")


@functools.partial(
    pl.kernel,
    out_type=jax.ShapeDtypeStruct((NC, NP, F), jnp.float32),
    mesh=_mesh,
    scratch_types=[
        pltpu.VMEM_SHARED((NP, F), jnp.float32),
        pltpu.VMEM((K, F), jnp.float32),
        pltpu.VMEM((K,), jnp.int32),
        pltpu.VMEM((K,), jnp.int32),
        pltpu.SemaphoreType.DMA,
        pltpu.SemaphoreType.DMA,
    ],
)
def _deg_kernel(dsta, dstb, dstc, zeros_hbm, m0, m1, m2, out,
                shd, mv, dv0, dv1, s0, s1):
    c = lax.axis_index("c")
    s = lax.axis_index("s")
    wid = _wid()
    dsts = [dsta, dstb, dstc]
    ms = [m0, m1, m2]
    dvs = [dv0, dv1]
    sems = [s0, s1]
    for j in range(5):
        cz = s + NS * j

        @pl.when(cz < NP // K)
        def _():
            pltpu.sync_copy(zeros_hbm, shd.at[pl.ds(cz * K, K)])

    plsc.subcore_barrier()
    base = wid * T
    for r in range(3):
        pltpu.sync_copy(ms[r], mv)
        pltpu.sync_copy(dsts[r].at[pl.ds(base * K, K)], dv0)

        def body(j0, carry):
            for b in range(2):
                j = 2 * j0 + b

                @pl.when(j >= 1)
                def _():
                    pltpu.make_async_copy(mv, shd.at[dvs[1 - b]],
                                          sems[1 - b]).wait()

                pltpu.async_copy(mv, shd.at[dvs[b]], sems[b], add=True)

                @pl.when(j + 1 < T)
                def _():
                    pltpu.sync_copy(
                        dsts[r].at[pl.ds((base + j + 1) * K, K)], dvs[1 - b])

            return carry

        lax.fori_loop(0, T // 2, body, 0)
        pltpu.make_async_copy(mv, shd.at[dvs[(T - 1) % 2]],
                              sems[(T - 1) % 2]).wait()
    plsc.subcore_barrier()
    pltpu.sync_copy(shd.at[pl.ds(s * ROWS_PER_SUB, ROWS_PER_SUB)],
                    out.at[c, pl.ds(s * ROWS_PER_SUB, ROWS_PER_SUB)])


@functools.partial(
    pl.kernel,
    out_type=jax.ShapeDtypeStruct((NC, 3, NP, F), jnp.float32),
    mesh=_mesh,
    scratch_types=[
        pltpu.VMEM_SHARED((NP, F), jnp.float32),
        pltpu.VMEM((K, F), jnp.float32),
        pltpu.VMEM((K, F), jnp.float32),
        pltpu.VMEM((K,), jnp.int32),
        pltpu.VMEM((K,), jnp.int32),
        pltpu.VMEM((K,), jnp.int32),
        pltpu.VMEM((K,), jnp.int32),
        pltpu.SemaphoreType.DMA,
        pltpu.SemaphoreType.DMA,
        pltpu.SemaphoreType.DMA,
        pltpu.SemaphoreType.DMA,
    ],
)
def _scatter_kernel(hp0, hp1, hp2, srca, srcb, srcc, dsta, dstb, dstc,
                    zeros_hbm, out, acc, rows0, rows1, sv0, sv1, dv0, dv1,
                    sg0, sg1, ss0, ss1):
    c = lax.axis_index("c")
    s = lax.axis_index("s")
    wid = _wid()
    hps = [hp0, hp1, hp2]
    srcs = [srca, srcb, srcc]
    dsts = [dsta, dstb, dstc]
    rows = [rows0, rows1]
    svs = [sv0, sv1]
    dvs = [dv0, dv1]
    sgs = [sg0, sg1]
    sss = [ss0, ss1]
    base = wid * T
    for r in range(3):
        for j in range(5):
            cz = s + NS * j

            @pl.when(cz < NP // K)
            def _():
                pltpu.sync_copy(zeros_hbm, acc.at[pl.ds(cz * K, K)])

        plsc.subcore_barrier()
        pltpu.sync_copy(srcs[r].at[pl.ds(base * K, K)], sv0)
        pltpu.sync_copy(dsts[r].at[pl.ds(base * K, K)], dv0)
        pltpu.async_copy(hps[r].at[sv0], rows0, sg0)

        def body(j0, carry):
            for b in range(2):
                j = 2 * j0 + b
                pltpu.make_async_copy(hps[r].at[svs[b]], rows[b],
                                      sgs[b]).wait()

                @pl.when(j + 1 < T)
                def _():
                    pltpu.sync_copy(
                        srcs[r].at[pl.ds((base + j + 1) * K, K)], svs[1 - b])
                    pltpu.sync_copy(
                        dsts[r].at[pl.ds((base + j + 1) * K, K)], dvs[1 - b])
                    pltpu.async_copy(hps[r].at[svs[1 - b]], rows[1 - b],
                                     sgs[1 - b])

                pltpu.sync_copy(rows[b], acc.at[dvs[b]], add=True)

            return carry

        lax.fori_loop(0, T // 2, body, 0)
        plsc.subcore_barrier()
        pltpu.sync_copy(acc.at[pl.ds(s * ROWS_PER_SUB, ROWS_PER_SUB)],
                        out.at[c, r, pl.ds(s * ROWS_PER_SUB, ROWS_PER_SUB)])
        if r < 2:
            plsc.subcore_barrier()


@functools.partial(
    pl.kernel,
    out_type=jax.ShapeDtypeStruct((B, F), jnp.float32),
    mesh=_mesh,
    scratch_types=[
        pltpu.VMEM((B // NW,), jnp.int32),
        pltpu.VMEM((B // NW, F), jnp.float32),
        pltpu.SemaphoreType.DMA,
    ],
)
def _gather_kernel(g_hbm, idx_hbm, out, idx_v, rows_v, sem):
    wid = _wid()
    base = wid * (B // NW)
    pltpu.sync_copy(idx_hbm.at[pl.ds(base, B // NW)], idx_v)
    pltpu.async_copy(g_hbm.at[idx_v], rows_v, sem).wait()
    pltpu.sync_copy(rows_v, out.at[pl.ds(base, B // NW)])


_BR = 1000


def _c1_body(x, dga, dgb, W0, W1, W2,
             hp0, hp1, hp2, dv0, dv1, dv2):
    Ws = [W0, W1, W2]
    hps = [hp0, hp1, hp2]
    dvs = [dv0, dv1, dv2]
    xb = x[...]
    for r in range(3):
        deg = dga[0, :, r:r + 1] + dgb[0, :, r:r + 1] + 1.0
        dinv = lax.rsqrt(deg)
        h = jnp.dot(xb, Ws[r][...], preferred_element_type=jnp.float32)
        hps[r][...] = h * dinv
        dvs[r][...] = dinv


def _c1(x, degp, W1s):
    spec_x = pl.BlockSpec((_BR, F), lambda i: (i, 0))
    spec_deg = [pl.BlockSpec((1, _BR, F), (lambda i, c=c: (c, i, 0)))
                for c in range(2)]
    spec_w = pl.BlockSpec((F, F), lambda i: (0, 0))
    spec_o = pl.BlockSpec((_BR, F), lambda i: (i, 0))
    spec_dv = pl.BlockSpec((_BR, 1), lambda i: (i, 0))
    return pl.pallas_call(
        _c1_body,
        grid=(N // _BR,),
        in_specs=[spec_x] + spec_deg + [spec_w] * 3,
        out_specs=[spec_o] * 3 + [spec_dv] * 3,
        out_shape=[jax.ShapeDtypeStruct((N, F), jnp.float32)] * 3
        + [jax.ShapeDtypeStruct((N, 1), jnp.float32)] * 3,
    )(x, degp, degp, *W1s)


def _ec_body(a00, a01, a10, a11, a20, a21, hp0, hp1, hp2,
             dv0, dv1, dv2, b0, b1, b2, W0, W1, W2, o0, o1, o2):
    accs = [(a00, a01), (a10, a11), (a20, a21)]
    hps = [hp0, hp1, hp2]
    dvs = [dv0, dv1, dv2]
    bs = [b0, b1, b2]
    Ws = [W0, W1, W2]
    outs = [o0, o1, o2]
    h = None
    for r in range(3):
        acc = accs[r][0][0, 0] + accs[r][1][0, 0]
        full = (acc + hps[r][...]) * dvs[r][...] + bs[r][...]
        v = jnp.maximum(full, 0.0)
        h = v if h is None else jnp.maximum(h, v)
    for r in range(3):
        o = jnp.dot(h, Ws[r][...], preferred_element_type=jnp.float32)
        outs[r][...] = o * dvs[r][...]


def _acc_specs():
    return [pl.BlockSpec((1, 1, _BR, F),
                         (lambda i, c=c, r=r: (c, r, i, 0)))
            for r in range(3) for c in range(2)]


def _ec(accp, hps, dvs, bs, W2s):
    spec_h = pl.BlockSpec((_BR, F), lambda i: (i, 0))
    spec_dv = pl.BlockSpec((_BR, 1), lambda i: (i, 0))
    spec_b = pl.BlockSpec((1, F), lambda i: (0, 0))
    spec_w = pl.BlockSpec((F, F), lambda i: (0, 0))
    return pl.pallas_call(
        _ec_body,
        grid=(N // _BR,),
        in_specs=_acc_specs() + [spec_h] * 3 + [spec_dv] * 3
        + [spec_b] * 3 + [spec_w] * 3,
        out_specs=[spec_h] * 3,
        out_shape=[jax.ShapeDtypeStruct((N, F), jnp.float32)] * 3,
    )(accp, accp, accp, accp, accp, accp, *hps, *dvs, *bs, *W2s)


def _e2_body(a00, a01, a10, a11, a20, a21, hp0, hp1, hp2,
             dv0, dv1, dv2, b0, b1, b2, g):
    accs = [(a00, a01), (a10, a11), (a20, a21)]
    hps = [hp0, hp1, hp2]
    dvs = [dv0, dv1, dv2]
    bs = [b0, b1, b2]
    h = None
    for r in range(3):
        acc = accs[r][0][0, 0] + accs[r][1][0, 0]
        full = (acc + hps[r][...]) * dvs[r][...] + bs[r][...]
        v = jnp.maximum(full, 0.0)
        h = v if h is None else jnp.maximum(h, v)
    g[...] = h


def _e2(accp, hps, dvs, bs):
    spec_h = pl.BlockSpec((_BR, F), lambda i: (i, 0))
    spec_dv = pl.BlockSpec((_BR, 1), lambda i: (i, 0))
    spec_b = pl.BlockSpec((1, F), lambda i: (0, 0))
    return pl.pallas_call(
        _e2_body,
        grid=(N // _BR,),
        in_specs=_acc_specs() + [spec_h] * 3 + [spec_dv] * 3 + [spec_b] * 3,
        out_specs=spec_h,
        out_shape=jax.ShapeDtypeStruct((N, F), jnp.float32),
    )(accp, accp, accp, accp, accp, accp, *hps, *dvs, *bs)


def kernel(x, syntactic_edge_index, sequential_edge_index, semantic_edge_index,
           index, W1_syn, b1_syn, W1_seq, b1_seq, W1_sem, b1_sem,
           W2_syn, b2_syn, W2_seq, b2_seq, W2_sem, b2_sem):
    es = [syntactic_edge_index, sequential_edge_index, semantic_edge_index]
    pad_src = jnp.arange(EPAD - E, dtype=jnp.int32) % N
    pad_dst = N + jnp.arange(EPAD - E, dtype=jnp.int32) % (NP - N)
    srcs = [jnp.concatenate([e[0], pad_src]) for e in es]
    dsts = [jnp.concatenate([e[1], pad_dst]) for e in es]
    zeros128 = jnp.zeros((K, F), jnp.float32)
    lane = lax.broadcasted_iota(jnp.int32, (K, F), 1)
    masks = [(lane == r).astype(jnp.float32) for r in range(3)]

    degp = _deg_kernel(dsts[0], dsts[1], dsts[2], zeros128, *masks)

    W1s = [W1_syn, W1_seq, W1_sem]
    b1s = [b1_syn.reshape(1, F), b1_seq.reshape(1, F), b1_sem.reshape(1, F)]
    W2s = [W2_syn, W2_seq, W2_sem]
    b2s = [b2_syn.reshape(1, F), b2_seq.reshape(1, F), b2_sem.reshape(1, F)]

    hp0, hp1, hp2, dv0, dv1, dv2 = _c1(x, degp, W1s)

    acc1 = _scatter_kernel(hp0, hp1, hp2, *srcs, *dsts, zeros128)

    h2p = _ec(acc1, [hp0, hp1, hp2], [dv0, dv1, dv2], b1s, W2s)

    acc2 = _scatter_kernel(h2p[0], h2p[1], h2p[2], *srcs, *dsts, zeros128)

    g = _e2(acc2, h2p, [dv0, dv1, dv2], b2s)
    return _gather_kernel(g, index)


# speedup vs baseline: 2.5201x; 1.0153x over previous
import functools

import jax
import jax.numpy as jnp
from jax import lax
from jax.experimental import pallas as pl
from jax.experimental.pallas import tpu as pltpu
from jax.experimental.pallas import tpu_sc as plsc

N = 10000
E = 320000
F = 128
B = 1024
K = 128
NCHUNK = E // K
NP = 10112
NC = 2
NS = 16
NW = NC * NS
T = 80
EPAD = T * NW * K
ROWS_PER_SUB = NP // NS

_mesh = plsc.VectorSubcoreMesh(core_axis_name="c", subcore_axis_name="s")


def _wid():
    return lax.axis_index("s") * NC + lax.axis_index("c")


@functools.partial(
    pl.kernel,
    out_type=jax.ShapeDtypeStruct((NC, NP, F), jnp.float32),
    mesh=_mesh,
    scratch_types=[
        pltpu.VMEM_SHARED((NP, F), jnp.float32),
        pltpu.VMEM((K, F), jnp.float32),
        pltpu.VMEM((K,), jnp.int32),
        pltpu.VMEM((K,), jnp.int32),
        pltpu.SemaphoreType.DMA,
        pltpu.SemaphoreType.DMA,
    ],
)
def _deg_kernel(dsta, dstb, dstc, zeros_hbm, m0, m1, m2, out,
                shd, mv, dv0, dv1, s0, s1):
    c = lax.axis_index("c")
    s = lax.axis_index("s")
    wid = _wid()
    dsts = [dsta, dstb, dstc]
    ms = [m0, m1, m2]
    dvs = [dv0, dv1]
    sems = [s0, s1]
    for j in range(5):
        cz = s + NS * j

        @pl.when(cz < NP // K)
        def _():
            pltpu.sync_copy(zeros_hbm, shd.at[pl.ds(cz * K, K)])

    plsc.subcore_barrier()
    base = wid * T
    for r in range(3):
        pltpu.sync_copy(ms[r], mv)
        pltpu.sync_copy(dsts[r].at[pl.ds(base * K, K)], dv0)

        def body(j0, carry):
            for b in range(2):
                j = 2 * j0 + b

                @pl.when(j >= 1)
                def _():
                    pltpu.make_async_copy(mv, shd.at[dvs[1 - b]],
                                          sems[1 - b]).wait()

                pltpu.async_copy(mv, shd.at[dvs[b]], sems[b], add=True)

                @pl.when(j + 1 < T)
                def _():
                    pltpu.sync_copy(
                        dsts[r].at[pl.ds((base + j + 1) * K, K)], dvs[1 - b])

            return carry

        lax.fori_loop(0, T // 2, body, 0)
        pltpu.make_async_copy(mv, shd.at[dvs[(T - 1) % 2]],
                              sems[(T - 1) % 2]).wait()
    plsc.subcore_barrier()
    pltpu.sync_copy(shd.at[pl.ds(s * ROWS_PER_SUB, ROWS_PER_SUB)],
                    out.at[c, pl.ds(s * ROWS_PER_SUB, ROWS_PER_SUB)])


@functools.partial(
    pl.kernel,
    out_type=jax.ShapeDtypeStruct((NC, 3, NP, F), jnp.float32),
    mesh=_mesh,
    scratch_types=[
        pltpu.VMEM_SHARED((NP, F), jnp.float32),
        pltpu.VMEM((K, F), jnp.float32),
        pltpu.VMEM((K, F), jnp.float32),
        pltpu.VMEM((K,), jnp.int32),
        pltpu.VMEM((K,), jnp.int32),
        pltpu.VMEM((K,), jnp.int32),
        pltpu.VMEM((K,), jnp.int32),
        pltpu.SemaphoreType.DMA,
        pltpu.SemaphoreType.DMA,
        pltpu.SemaphoreType.DMA,
        pltpu.SemaphoreType.DMA,
    ],
)
def _scatter_kernel(hp0, hp1, hp2, srca, srcb, srcc, dsta, dstb, dstc,
                    zeros_hbm, out, acc, rows0, rows1, sv0, sv1, dv0, dv1,
                    sg0, sg1, ss0, ss1):
    c = lax.axis_index("c")
    s = lax.axis_index("s")
    wid = _wid()
    hps = [hp0, hp1, hp2]
    srcs = [srca, srcb, srcc]
    dsts = [dsta, dstb, dstc]
    rows = [rows0, rows1]
    svs = [sv0, sv1]
    dvs = [dv0, dv1]
    sgs = [sg0, sg1]
    sss = [ss0, ss1]
    base = wid * T
    for r in range(3):
        for j in range(5):
            cz = s + NS * j

            @pl.when(cz < NP // K)
            def _():
                pltpu.sync_copy(zeros_hbm, acc.at[pl.ds(cz * K, K)])

        plsc.subcore_barrier()
        pltpu.sync_copy(srcs[r].at[pl.ds(base * K, K)], sv0)
        pltpu.sync_copy(dsts[r].at[pl.ds(base * K, K)], dv0)
        pltpu.async_copy(hps[r].at[sv0], rows0, sg0)

        def body(j0, carry):
            for b in range(2):
                j = 2 * j0 + b
                pltpu.make_async_copy(hps[r].at[svs[b]], rows[b],
                                      sgs[b]).wait()

                @pl.when(j >= 1)
                def _():
                    pltpu.make_async_copy(rows[1 - b], acc.at[dvs[1 - b]],
                                          sss[1 - b]).wait()

                pltpu.async_copy(rows[b], acc.at[dvs[b]], sss[b], add=True)

                @pl.when(j + 1 < T)
                def _():
                    pltpu.sync_copy(
                        srcs[r].at[pl.ds((base + j + 1) * K, K)], svs[1 - b])
                    pltpu.sync_copy(
                        dsts[r].at[pl.ds((base + j + 1) * K, K)], dvs[1 - b])
                    pltpu.async_copy(hps[r].at[svs[1 - b]], rows[1 - b],
                                     sgs[1 - b])

            return carry

        lax.fori_loop(0, T // 2, body, 0)
        pltpu.make_async_copy(rows[(T - 1) % 2], acc.at[dvs[(T - 1) % 2]],
                              sss[(T - 1) % 2]).wait()
        plsc.subcore_barrier()
        pltpu.sync_copy(acc.at[pl.ds(s * ROWS_PER_SUB, ROWS_PER_SUB)],
                        out.at[c, r, pl.ds(s * ROWS_PER_SUB, ROWS_PER_SUB)])
        if r < 2:
            plsc.subcore_barrier()


@functools.partial(
    pl.kernel,
    out_type=jax.ShapeDtypeStruct((B, F), jnp.float32),
    mesh=_mesh,
    scratch_types=[
        pltpu.VMEM((B // NW,), jnp.int32),
        pltpu.VMEM((B // NW, F), jnp.float32),
        pltpu.SemaphoreType.DMA,
    ],
)
def _gather_kernel(g_hbm, idx_hbm, out, idx_v, rows_v, sem):
    wid = _wid()
    base = wid * (B // NW)
    pltpu.sync_copy(idx_hbm.at[pl.ds(base, B // NW)], idx_v)
    pltpu.async_copy(g_hbm.at[idx_v], rows_v, sem).wait()
    pltpu.sync_copy(rows_v, out.at[pl.ds(base, B // NW)])


_BR = 1000


def _c1_body(x, dga, dgb, W0, W1, W2,
             hp0, hp1, hp2, dv0, dv1, dv2):
    Ws = [W0, W1, W2]
    hps = [hp0, hp1, hp2]
    dvs = [dv0, dv1, dv2]
    xb = x[...]
    for r in range(3):
        deg = dga[0, :, r:r + 1] + dgb[0, :, r:r + 1] + 1.0
        dinv = lax.rsqrt(deg)
        h = jnp.dot(xb, Ws[r][...], preferred_element_type=jnp.float32)
        hps[r][...] = h * dinv
        dvs[r][...] = dinv


def _c1(x, degp, W1s):
    spec_x = pl.BlockSpec((_BR, F), lambda i: (i, 0))
    spec_deg = [pl.BlockSpec((1, _BR, F), (lambda i, c=c: (c, i, 0)))
                for c in range(2)]
    spec_w = pl.BlockSpec((F, F), lambda i: (0, 0))
    spec_o = pl.BlockSpec((_BR, F), lambda i: (i, 0))
    spec_dv = pl.BlockSpec((_BR, 1), lambda i: (i, 0))
    return pl.pallas_call(
        _c1_body,
        grid=(N // _BR,),
        in_specs=[spec_x] + spec_deg + [spec_w] * 3,
        out_specs=[spec_o] * 3 + [spec_dv] * 3,
        out_shape=[jax.ShapeDtypeStruct((N, F), jnp.float32)] * 3
        + [jax.ShapeDtypeStruct((N, 1), jnp.float32)] * 3,
    )(x, degp, degp, *W1s)


def _ec_body(a00, a01, a10, a11, a20, a21, hp0, hp1, hp2,
             dv0, dv1, dv2, b0, b1, b2, W0, W1, W2, o0, o1, o2):
    accs = [(a00, a01), (a10, a11), (a20, a21)]
    hps = [hp0, hp1, hp2]
    dvs = [dv0, dv1, dv2]
    bs = [b0, b1, b2]
    Ws = [W0, W1, W2]
    outs = [o0, o1, o2]
    h = None
    for r in range(3):
        acc = accs[r][0][0, 0] + accs[r][1][0, 0]
        full = (acc + hps[r][...]) * dvs[r][...] + bs[r][...]
        v = jnp.maximum(full, 0.0)
        h = v if h is None else jnp.maximum(h, v)
    for r in range(3):
        o = jnp.dot(h, Ws[r][...], preferred_element_type=jnp.float32)
        outs[r][...] = o * dvs[r][...]


def _acc_specs():
    return [pl.BlockSpec((1, 1, _BR, F),
                         (lambda i, c=c, r=r: (c, r, i, 0)))
            for r in range(3) for c in range(2)]


def _ec(accp, hps, dvs, bs, W2s):
    spec_h = pl.BlockSpec((_BR, F), lambda i: (i, 0))
    spec_dv = pl.BlockSpec((_BR, 1), lambda i: (i, 0))
    spec_b = pl.BlockSpec((1, F), lambda i: (0, 0))
    spec_w = pl.BlockSpec((F, F), lambda i: (0, 0))
    return pl.pallas_call(
        _ec_body,
        grid=(N // _BR,),
        in_specs=_acc_specs() + [spec_h] * 3 + [spec_dv] * 3
        + [spec_b] * 3 + [spec_w] * 3,
        out_specs=[spec_h] * 3,
        out_shape=[jax.ShapeDtypeStruct((N, F), jnp.float32)] * 3,
    )(accp, accp, accp, accp, accp, accp, *hps, *dvs, *bs, *W2s)


def _e2_body(a00, a01, a10, a11, a20, a21, hp0, hp1, hp2,
             dv0, dv1, dv2, b0, b1, b2, g):
    accs = [(a00, a01), (a10, a11), (a20, a21)]
    hps = [hp0, hp1, hp2]
    dvs = [dv0, dv1, dv2]
    bs = [b0, b1, b2]
    h = None
    for r in range(3):
        acc = accs[r][0][0, 0] + accs[r][1][0, 0]
        full = (acc + hps[r][...]) * dvs[r][...] + bs[r][...]
        v = jnp.maximum(full, 0.0)
        h = v if h is None else jnp.maximum(h, v)
    g[...] = h


def _e2(accp, hps, dvs, bs):
    spec_h = pl.BlockSpec((_BR, F), lambda i: (i, 0))
    spec_dv = pl.BlockSpec((_BR, 1), lambda i: (i, 0))
    spec_b = pl.BlockSpec((1, F), lambda i: (0, 0))
    return pl.pallas_call(
        _e2_body,
        grid=(N // _BR,),
        in_specs=_acc_specs() + [spec_h] * 3 + [spec_dv] * 3 + [spec_b] * 3,
        out_specs=spec_h,
        out_shape=jax.ShapeDtypeStruct((N, F), jnp.float32),
    )(accp, accp, accp, accp, accp, accp, *hps, *dvs, *bs)


def kernel(x, syntactic_edge_index, sequential_edge_index, semantic_edge_index,
           index, W1_syn, b1_syn, W1_seq, b1_seq, W1_sem, b1_sem,
           W2_syn, b2_syn, W2_seq, b2_seq, W2_sem, b2_sem):
    es = [syntactic_edge_index, sequential_edge_index, semantic_edge_index]
    pad_src = jnp.arange(EPAD - E, dtype=jnp.int32) % N
    pad_dst = N + jnp.arange(EPAD - E, dtype=jnp.int32) % (NP - N)
    srcs = [jnp.concatenate([e[0], pad_src]) for e in es]
    dsts = [jnp.concatenate([e[1], pad_dst]) for e in es]
    zeros128 = jnp.zeros((K, F), jnp.float32)
    lane = lax.broadcasted_iota(jnp.int32, (K, F), 1)
    masks = [(lane == r).astype(jnp.float32) for r in range(3)]

    degp = _deg_kernel(dsts[0], dsts[1], dsts[2], zeros128, *masks)

    W1s = [W1_syn, W1_seq, W1_sem]
    b1s = [b1_syn.reshape(1, F), b1_seq.reshape(1, F), b1_sem.reshape(1, F)]
    W2s = [W2_syn, W2_seq, W2_sem]
    b2s = [b2_syn.reshape(1, F), b2_seq.reshape(1, F), b2_sem.reshape(1, F)]

    hp0, hp1, hp2, dv0, dv1, dv2 = _c1(x, degp, W1s)

    acc1 = _scatter_kernel(hp0, hp1, hp2, *srcs, *dsts, zeros128)

    h2p = _ec(acc1, [hp0, hp1, hp2], [dv0, dv1, dv2], b1s, W2s)

    acc2 = _scatter_kernel(h2p[0], h2p[1], h2p[2], *srcs, *dsts, zeros128)

    g = _e2(acc2, h2p, [dv0, dv1, dv2], b2s)
    return _gather_kernel(g, index)


# speedup vs baseline: 2.8885x; 1.1462x over previous
import functools

import jax
import jax.numpy as jnp
from jax import lax
from jax.experimental import pallas as pl
from jax.experimental.pallas import tpu as pltpu
from jax.experimental.pallas import tpu_sc as plsc

N = 10000
E = 320000
F = 128
B = 1024
K = 128
NCHUNK = E // K
NP = 10112
NC = 2
NS = 16
NW = NC * NS
T = 80
EPAD = T * NW * K
ROWS_PER_SUB = NP // NS

_mesh = plsc.VectorSubcoreMesh(core_axis_name="c", subcore_axis_name="s")


def _wid():
    return lax.axis_index("s") * NC + lax.axis_index("c")


@functools.partial(
    pl.kernel,
    out_type=jax.ShapeDtypeStruct((NC, NP, F), jnp.float32),
    mesh=_mesh,
    scratch_types=[
        pltpu.VMEM_SHARED((NP, F), jnp.float32),
        pltpu.VMEM((K, F), jnp.float32),
        pltpu.VMEM((K,), jnp.int32),
        pltpu.VMEM((K,), jnp.int32),
        pltpu.SemaphoreType.DMA,
        pltpu.SemaphoreType.DMA,
    ],
)
def _deg_kernel(dsta, dstb, dstc, zeros_hbm, m0, m1, m2, out,
                shd, mv, dv0, dv1, s0, s1):
    c = lax.axis_index("c")
    s = lax.axis_index("s")
    wid = _wid()
    dsts = [dsta, dstb, dstc]
    ms = [m0, m1, m2]
    dvs = [dv0, dv1]
    sems = [s0, s1]
    for j in range(5):
        cz = s + NS * j

        @pl.when(cz < NP // K)
        def _():
            pltpu.sync_copy(zeros_hbm, shd.at[pl.ds(cz * K, K)])

    plsc.subcore_barrier()
    base = wid * T
    for r in range(3):
        pltpu.sync_copy(ms[r], mv)
        pltpu.sync_copy(dsts[r].at[pl.ds(base * K, K)], dv0)

        def body(j0, carry):
            for b in range(2):
                j = 2 * j0 + b

                @pl.when(j >= 1)
                def _():
                    pltpu.make_async_copy(mv, shd.at[dvs[1 - b]],
                                          sems[1 - b]).wait()

                pltpu.async_copy(mv, shd.at[dvs[b]], sems[b], add=True)

                @pl.when(j + 1 < T)
                def _():
                    pltpu.sync_copy(
                        dsts[r].at[pl.ds((base + j + 1) * K, K)], dvs[1 - b])

            return carry

        lax.fori_loop(0, T // 2, body, 0)
        pltpu.make_async_copy(mv, shd.at[dvs[(T - 1) % 2]],
                              sems[(T - 1) % 2]).wait()
    plsc.subcore_barrier()
    pltpu.sync_copy(shd.at[pl.ds(s * ROWS_PER_SUB, ROWS_PER_SUB)],
                    out.at[c, pl.ds(s * ROWS_PER_SUB, ROWS_PER_SUB)])


@functools.partial(
    pl.kernel,
    out_type=jax.ShapeDtypeStruct((NC, 3, NP, F), jnp.float32),
    mesh=_mesh,
    scratch_types=[
        pltpu.VMEM_SHARED((NP, F), jnp.float32),
        pltpu.VMEM((K, F), jnp.float32),
        pltpu.VMEM((K, F), jnp.float32),
        pltpu.VMEM((T * K,), jnp.int32),
        pltpu.VMEM((K,), jnp.int32),
        pltpu.VMEM((K,), jnp.int32),
        pltpu.SemaphoreType.DMA,
        pltpu.SemaphoreType.DMA,
        pltpu.SemaphoreType.DMA,
        pltpu.SemaphoreType.DMA,
    ],
)
def _scatter_kernel(hp0, hp1, hp2, srca, srcb, srcc, dsta, dstb, dstc,
                    zeros_hbm, out, acc, rows0, rows1, svb, dv0, dv1,
                    sg0, sg1, ss0, ss1):
    c = lax.axis_index("c")
    s = lax.axis_index("s")
    wid = _wid()
    hps = [hp0, hp1, hp2]
    srcs = [srca, srcb, srcc]
    dsts = [dsta, dstb, dstc]
    rows = [rows0, rows1]
    dvs = [dv0, dv1]
    sgs = [sg0, sg1]
    sss = [ss0, ss1]
    base = wid * T
    for r in range(3):
        for j in range(5):
            cz = s + NS * j

            @pl.when(cz < NP // K)
            def _():
                pltpu.sync_copy(zeros_hbm, acc.at[pl.ds(cz * K, K)])

        plsc.subcore_barrier()
        pltpu.sync_copy(srcs[r].at[pl.ds(base * K, T * K)], svb)
        pltpu.sync_copy(dsts[r].at[pl.ds(base * K, K)], dv0)
        pltpu.async_copy(hps[r].at[svb.at[pl.ds(0, K)]], rows0, sg0)

        def body(j0, carry):
            for b in range(2):
                j = 2 * j0 + b
                pltpu.make_async_copy(hps[r].at[svb.at[pl.ds(0, K)]], rows[b],
                                      sgs[b]).wait()

                @pl.when(j >= 1)
                def _():
                    pltpu.make_async_copy(rows[1 - b], acc.at[dvs[1 - b]],
                                          sss[1 - b]).wait()

                pltpu.async_copy(rows[b], acc.at[dvs[b]], sss[b], add=True)

                @pl.when(j + 1 < T)
                def _():
                    pltpu.sync_copy(
                        dsts[r].at[pl.ds((base + j + 1) * K, K)], dvs[1 - b])
                    pltpu.async_copy(
                        hps[r].at[svb.at[pl.ds((j + 1) * K, K)]], rows[1 - b],
                        sgs[1 - b])

            return carry

        lax.fori_loop(0, T // 2, body, 0)
        pltpu.make_async_copy(rows[(T - 1) % 2], acc.at[dvs[(T - 1) % 2]],
                              sss[(T - 1) % 2]).wait()
        plsc.subcore_barrier()
        pltpu.sync_copy(acc.at[pl.ds(s * ROWS_PER_SUB, ROWS_PER_SUB)],
                        out.at[c, r, pl.ds(s * ROWS_PER_SUB, ROWS_PER_SUB)])
        if r < 2:
            plsc.subcore_barrier()


@functools.partial(
    pl.kernel,
    out_type=jax.ShapeDtypeStruct((B, F), jnp.float32),
    mesh=_mesh,
    scratch_types=[
        pltpu.VMEM((B // NW,), jnp.int32),
        pltpu.VMEM((B // NW, F), jnp.float32),
        pltpu.SemaphoreType.DMA,
    ],
)
def _gather_kernel(g_hbm, idx_hbm, out, idx_v, rows_v, sem):
    wid = _wid()
    base = wid * (B // NW)
    pltpu.sync_copy(idx_hbm.at[pl.ds(base, B // NW)], idx_v)
    pltpu.async_copy(g_hbm.at[idx_v], rows_v, sem).wait()
    pltpu.sync_copy(rows_v, out.at[pl.ds(base, B // NW)])


_BR = 1000


def _c1_body(x, dga, dgb, W0, W1, W2,
             hp0, hp1, hp2, dv0, dv1, dv2):
    Ws = [W0, W1, W2]
    hps = [hp0, hp1, hp2]
    dvs = [dv0, dv1, dv2]
    xb = x[...]
    for r in range(3):
        deg = dga[0, :, r:r + 1] + dgb[0, :, r:r + 1] + 1.0
        dinv = lax.rsqrt(deg)
        h = jnp.dot(xb, Ws[r][...], preferred_element_type=jnp.float32)
        hps[r][...] = h * dinv
        dvs[r][...] = dinv


def _c1(x, degp, W1s):
    spec_x = pl.BlockSpec((_BR, F), lambda i: (i, 0))
    spec_deg = [pl.BlockSpec((1, _BR, F), (lambda i, c=c: (c, i, 0)))
                for c in range(2)]
    spec_w = pl.BlockSpec((F, F), lambda i: (0, 0))
    spec_o = pl.BlockSpec((_BR, F), lambda i: (i, 0))
    spec_dv = pl.BlockSpec((_BR, 1), lambda i: (i, 0))
    return pl.pallas_call(
        _c1_body,
        grid=(N // _BR,),
        in_specs=[spec_x] + spec_deg + [spec_w] * 3,
        out_specs=[spec_o] * 3 + [spec_dv] * 3,
        out_shape=[jax.ShapeDtypeStruct((N, F), jnp.float32)] * 3
        + [jax.ShapeDtypeStruct((N, 1), jnp.float32)] * 3,
    )(x, degp, degp, *W1s)


def _ec_body(a00, a01, a10, a11, a20, a21, hp0, hp1, hp2,
             dv0, dv1, dv2, b0, b1, b2, W0, W1, W2, o0, o1, o2):
    accs = [(a00, a01), (a10, a11), (a20, a21)]
    hps = [hp0, hp1, hp2]
    dvs = [dv0, dv1, dv2]
    bs = [b0, b1, b2]
    Ws = [W0, W1, W2]
    outs = [o0, o1, o2]
    h = None
    for r in range(3):
        acc = accs[r][0][0, 0] + accs[r][1][0, 0]
        full = (acc + hps[r][...]) * dvs[r][...] + bs[r][...]
        v = jnp.maximum(full, 0.0)
        h = v if h is None else jnp.maximum(h, v)
    for r in range(3):
        o = jnp.dot(h, Ws[r][...], preferred_element_type=jnp.float32)
        outs[r][...] = o * dvs[r][...]


def _acc_specs():
    return [pl.BlockSpec((1, 1, _BR, F),
                         (lambda i, c=c, r=r: (c, r, i, 0)))
            for r in range(3) for c in range(2)]


def _ec(accp, hps, dvs, bs, W2s):
    spec_h = pl.BlockSpec((_BR, F), lambda i: (i, 0))
    spec_dv = pl.BlockSpec((_BR, 1), lambda i: (i, 0))
    spec_b = pl.BlockSpec((1, F), lambda i: (0, 0))
    spec_w = pl.BlockSpec((F, F), lambda i: (0, 0))
    return pl.pallas_call(
        _ec_body,
        grid=(N // _BR,),
        in_specs=_acc_specs() + [spec_h] * 3 + [spec_dv] * 3
        + [spec_b] * 3 + [spec_w] * 3,
        out_specs=[spec_h] * 3,
        out_shape=[jax.ShapeDtypeStruct((N, F), jnp.float32)] * 3,
    )(accp, accp, accp, accp, accp, accp, *hps, *dvs, *bs, *W2s)


def _e2_body(a00, a01, a10, a11, a20, a21, hp0, hp1, hp2,
             dv0, dv1, dv2, b0, b1, b2, g):
    accs = [(a00, a01), (a10, a11), (a20, a21)]
    hps = [hp0, hp1, hp2]
    dvs = [dv0, dv1, dv2]
    bs = [b0, b1, b2]
    h = None
    for r in range(3):
        acc = accs[r][0][0, 0] + accs[r][1][0, 0]
        full = (acc + hps[r][...]) * dvs[r][...] + bs[r][...]
        v = jnp.maximum(full, 0.0)
        h = v if h is None else jnp.maximum(h, v)
    g[...] = h


def _e2(accp, hps, dvs, bs):
    spec_h = pl.BlockSpec((_BR, F), lambda i: (i, 0))
    spec_dv = pl.BlockSpec((_BR, 1), lambda i: (i, 0))
    spec_b = pl.BlockSpec((1, F), lambda i: (0, 0))
    return pl.pallas_call(
        _e2_body,
        grid=(N // _BR,),
        in_specs=_acc_specs() + [spec_h] * 3 + [spec_dv] * 3 + [spec_b] * 3,
        out_specs=spec_h,
        out_shape=jax.ShapeDtypeStruct((N, F), jnp.float32),
    )(accp, accp, accp, accp, accp, accp, *hps, *dvs, *bs)


def kernel(x, syntactic_edge_index, sequential_edge_index, semantic_edge_index,
           index, W1_syn, b1_syn, W1_seq, b1_seq, W1_sem, b1_sem,
           W2_syn, b2_syn, W2_seq, b2_seq, W2_sem, b2_sem):
    es = [syntactic_edge_index, sequential_edge_index, semantic_edge_index]
    pad_src = jnp.arange(EPAD - E, dtype=jnp.int32) % N
    pad_dst = N + jnp.arange(EPAD - E, dtype=jnp.int32) % (NP - N)
    srcs = [jnp.concatenate([e[0], pad_src]) for e in es]
    dsts = [jnp.concatenate([e[1], pad_dst]) for e in es]
    zeros128 = jnp.zeros((K, F), jnp.float32)
    lane = lax.broadcasted_iota(jnp.int32, (K, F), 1)
    masks = [(lane == r).astype(jnp.float32) for r in range(3)]

    degp = _deg_kernel(dsts[0], dsts[1], dsts[2], zeros128, *masks)

    W1s = [W1_syn, W1_seq, W1_sem]
    b1s = [b1_syn.reshape(1, F), b1_seq.reshape(1, F), b1_sem.reshape(1, F)]
    W2s = [W2_syn, W2_seq, W2_sem]
    b2s = [b2_syn.reshape(1, F), b2_seq.reshape(1, F), b2_sem.reshape(1, F)]

    hp0, hp1, hp2, dv0, dv1, dv2 = _c1(x, degp, W1s)

    acc1 = _scatter_kernel(hp0, hp1, hp2, *srcs, *dsts, zeros128)

    h2p = _ec(acc1, [hp0, hp1, hp2], [dv0, dv1, dv2], b1s, W2s)

    acc2 = _scatter_kernel(h2p[0], h2p[1], h2p[2], *srcs, *dsts, zeros128)

    g = _e2(acc2, h2p, [dv0, dv1, dv2], b2s)
    return _gather_kernel(g, index)
